# baseline (speedup 1.0000x reference)
import functools

import jax
import jax.numpy as jnp
from jax import lax
from jax.experimental import pallas as pl
from jax.experimental.pallas import tpu as pltpu
from jax.experimental.pallas import tpu_sc as plsc

EMBED_DIM = 32
BATCH = 16384
FIELDS = 26
B = BATCH * FIELDS

NC, NS = 2, 16
NW = NC * NS
BPW = B // NW
G = 1024
NG = BPW // G
GPF = BATCH // (G // 128 * 128)

_mesh = plsc.VectorSubcoreMesh(core_axis_name="c", subcore_axis_name="s")


@functools.partial(
    pl.kernel,
    out_type=jax.ShapeDtypeStruct((FIELDS, 4, 128, 8, 128), jnp.float32),
    mesh=_mesh,
    scratch_types=[
        pltpu.VMEM((NG, G), jnp.int32),
        pltpu.VMEM((G, EMBED_DIM), jnp.float32),
        pltpu.VMEM((4, 8, 8, 128), jnp.float32),
        pltpu.SemaphoreType.DMA,
    ],
    compiler_params=pltpu.CompilerParams(
        use_tc_tiling_on_sc=False, needs_layout_passes=False),
)
def _embed(table_hbm, idx_hbm, out_hbm, idx_v, rows_v, tile_v, sem):
    wid = lax.axis_index("s") * NC + lax.axis_index("c")
    pltpu.sync_copy(idx_hbm.at[wid], idx_v)
    lane = lax.iota(jnp.int32, 16)

    def body(g, _):
        gid = wid * NG + g
        f = gid // GPF
        bc0 = (gid % GPF) * 8
        pltpu.async_copy(table_hbm.at[idx_v.at[g]], rows_v, sem).wait()
        def bcl_body(bcl, _):
            row0 = bcl * 128
            for lg in range(8):
                rows = row0 + lg * 16 + lane
                for e in range(EMBED_DIM):
                    v = plsc.load_gather(
                        rows_v, [rows, jnp.full((16,), e, jnp.int32)])
                    tile_v[e // 8, bcl, e % 8, pl.ds(lg * 16, 16)] = v
            return 0
        lax.fori_loop(0, 8, bcl_body, 0)
        pltpu.sync_copy(tile_v, out_hbm.at[f, :, pl.ds(bc0, 8)])
        return 0

    lax.fori_loop(0, NG, body, 0)


def kernel(x, table):
    idx = x.T.reshape(NW, NG, G).astype(jnp.int32)
    out = _embed(table, idx)
    out = out.transpose(2, 4, 0, 1, 3).reshape(BATCH, FIELDS, EMBED_DIM)
    return out


# speedup vs baseline: 1.1756x; 1.1756x over previous
import functools

import jax
import jax.numpy as jnp
from jax import lax
from jax.experimental import pallas as pl
from jax.experimental.pallas import tpu as pltpu
from jax.experimental.pallas import tpu_sc as plsc

EMBED_DIM = 32
BATCH = 16384
FIELDS = 26
B = BATCH * FIELDS

NC, NS = 2, 16
NW = NC * NS
BPW = B // NW
CH = 1664
NCH = BPW // CH
COLS = BATCH // NW

_mesh = plsc.VectorSubcoreMesh(core_axis_name="c", subcore_axis_name="s")


@functools.partial(
    pl.kernel,
    out_type=jax.ShapeDtypeStruct((B,), jnp.int32),
    mesh=_mesh,
    scratch_types=[
        pltpu.VMEM((FIELDS, COLS), jnp.int32),
        pltpu.SemaphoreType.DMA,
    ],
    compiler_params=pltpu.CompilerParams(
        use_tc_tiling_on_sc=True, needs_layout_passes=False),
)
def _fmt(xt_hbm, out_hbm, buf, sem):
    wid = lax.axis_index("s") * NC + lax.axis_index("c")
    c0 = wid * COLS
    pltpu.sync_copy(xt_hbm.at[:, pl.ds(c0, COLS)], buf)
    for f in range(FIELDS):
        pltpu.async_copy(
            buf.at[f], out_hbm.at[pl.ds(f * BATCH + c0, COLS)], sem)
    for f in range(FIELDS):
        pltpu.make_async_copy(
            buf.at[f], out_hbm.at[pl.ds(f * BATCH + c0, COLS)], sem).wait()


@functools.partial(
    pl.kernel,
    out_type=jax.ShapeDtypeStruct((B, EMBED_DIM), jnp.float32),
    mesh=_mesh,
    scratch_types=[
        pltpu.VMEM((2, CH), jnp.int32),
        pltpu.VMEM((2, CH, EMBED_DIM), jnp.float32),
        pltpu.SemaphoreType.DMA,
        pltpu.SemaphoreType.DMA,
        pltpu.SemaphoreType.DMA,
        pltpu.SemaphoreType.DMA,
        pltpu.SemaphoreType.DMA,
        pltpu.SemaphoreType.DMA,
    ],
    compiler_params=pltpu.CompilerParams(use_tc_tiling_on_sc=False),
)
def _embed(table_hbm, idx_hbm, out_hbm, idx_v, rows_v,
           si0, si1, sg0, sg1, ss0, ss1):
    wid = lax.axis_index("s") * NC + lax.axis_index("c")
    base = wid * BPW
    si = (si0, si1)
    sg = (sg0, sg1)
    ss = (ss0, ss1)

    def idx_copy(i):
        b = i % 2
        return pltpu.make_async_copy(
            idx_hbm.at[pl.ds(base + i * CH, CH)], idx_v.at[b], si[b])

    def gather_copy(i):
        b = i % 2
        return pltpu.make_async_copy(
            table_hbm.at[idx_v.at[b]], rows_v.at[b], sg[b])

    def store_copy(i):
        b = i % 2
        return pltpu.make_async_copy(
            rows_v.at[b], out_hbm.at[pl.ds(base + i * CH, CH)], ss[b])

    idx_copy(0).start()
    idx_copy(1).start()
    for i in range(NCH):
        idx_copy(i).wait()
        if i >= 2:
            store_copy(i - 2).wait()
        gather_copy(i).start()
        if i >= 1:
            gather_copy(i - 1).wait()
            store_copy(i - 1).start()
            if 2 <= i + 1 < NCH:
                idx_copy(i + 1).start()
    gather_copy(NCH - 1).wait()
    store_copy(NCH - 1).start()
    store_copy(NCH - 2).wait()
    store_copy(NCH - 1).wait()


def kernel(x, table):
    idx = _fmt(x.T.astype(jnp.int32))
    out = _embed(table, idx)
    return out.reshape(FIELDS, BATCH, EMBED_DIM).transpose(1, 0, 2)


# speedup vs baseline: 1.1813x; 1.0048x over previous
import functools

import jax
import jax.numpy as jnp
from jax import lax
from jax.experimental import pallas as pl
from jax.experimental.pallas import tpu as pltpu
from jax.experimental.pallas import tpu_sc as plsc

INP = 1000000
EMBED_DIM = 32
BATCH = 16384
FIELDS = 26
B = BATCH * FIELDS

NC, NS = 2, 16
NW = NC * NS
BPW = B // NW
CH = 1664
NCH = BPW // CH
COLS = BATCH // NW

_mesh = plsc.VectorSubcoreMesh(core_axis_name="c", subcore_axis_name="s")


@functools.partial(
    pl.kernel,
    out_type=jax.ShapeDtypeStruct((B,), jnp.int32),
    mesh=_mesh,
    scratch_types=[
        pltpu.VMEM((FIELDS, COLS), jnp.int32),
        pltpu.SemaphoreType.DMA,
    ],
    compiler_params=pltpu.CompilerParams(
        use_tc_tiling_on_sc=True, needs_layout_passes=False),
)
def _fmt(xt_hbm, out_hbm, buf, sem):
    wid = lax.axis_index("s") * NC + lax.axis_index("c")
    c0 = wid * COLS
    pltpu.sync_copy(xt_hbm.at[:, pl.ds(c0, COLS)], buf)
    for f in range(FIELDS):
        pltpu.async_copy(
            buf.at[f], out_hbm.at[pl.ds(f * BATCH + c0, COLS)], sem)
    for f in range(FIELDS):
        pltpu.make_async_copy(
            buf.at[f], out_hbm.at[pl.ds(f * BATCH + c0, COLS)], sem).wait()


@functools.partial(
    pl.kernel,
    out_type=jax.ShapeDtypeStruct((B, EMBED_DIM), jnp.float32),
    mesh=_mesh,
    scratch_types=[
        pltpu.VMEM((2, CH), jnp.int32),
        pltpu.VMEM((2, CH, EMBED_DIM), jnp.float32),
        pltpu.SemaphoreType.DMA,
        pltpu.SemaphoreType.DMA,
        pltpu.SemaphoreType.DMA,
        pltpu.SemaphoreType.DMA,
        pltpu.SemaphoreType.DMA,
        pltpu.SemaphoreType.DMA,
    ],
    compiler_params=pltpu.CompilerParams(use_tc_tiling_on_sc=False),
)
def _embed(table_hbm, idx_hbm, out_hbm, idx_v, rows_v,
           si0, si1, sg0, sg1, ss0, ss1):
    wid = lax.axis_index("s") * NC + lax.axis_index("c")
    base = wid * BPW
    si = (si0, si1)
    sg = (sg0, sg1)
    ss = (ss0, ss1)

    def idx_copy(i):
        b = i % 2
        return pltpu.make_async_copy(
            idx_hbm.at[pl.ds(base + i * CH, CH)], idx_v.at[b], si[b])

    def gather_copy(i):
        b = i % 2
        return pltpu.make_async_copy(
            table_hbm.at[idx_v.at[b]], rows_v.at[b], sg[b])

    def store_copy(i):
        b = i % 2
        return pltpu.make_async_copy(
            rows_v.at[b], out_hbm.at[pl.ds(base + i * CH, CH)], ss[b])

    idx_copy(0).start()
    idx_copy(1).start()
    for i in range(NCH):
        idx_copy(i).wait()
        if i >= 2:
            store_copy(i - 2).wait()
        gather_copy(i).start()
        if i >= 1:
            gather_copy(i - 1).wait()
            store_copy(i - 1).start()
            if 2 <= i + 1 < NCH:
                idx_copy(i + 1).start()
    gather_copy(NCH - 1).wait()
    store_copy(NCH - 1).start()
    store_copy(NCH - 2).wait()
    store_copy(NCH - 1).wait()


def kernel(x, table):
    t128 = jax.lax.optimization_barrier(table.reshape(250000, 128))
    tbl = t128.reshape(INP, EMBED_DIM)
    idx = _fmt(x.T.astype(jnp.int32))
    out = _embed(tbl, idx)
    return out.reshape(FIELDS, BATCH, EMBED_DIM).transpose(1, 0, 2)


# speedup vs baseline: 1.5256x; 1.2915x over previous
import functools

import jax
import jax.numpy as jnp
from jax import lax
from jax.experimental import pallas as pl
from jax.experimental.pallas import tpu as pltpu
from jax.experimental.pallas import tpu_sc as plsc

INP = 1000000
EMBED_DIM = 32
BATCH = 16384
FIELDS = 26
B = BATCH * FIELDS

NC, NS = 2, 16
NW = NC * NS
BPW = B // NW
CH = 1664
NCH = BPW // CH
COLS = BATCH // NW

_mesh = plsc.VectorSubcoreMesh(core_axis_name="c", subcore_axis_name="s")


@functools.partial(
    pl.kernel,
    out_type=jax.ShapeDtypeStruct((B,), jnp.int32),
    mesh=_mesh,
    scratch_types=[
        pltpu.VMEM((FIELDS, COLS), jnp.int32),
        pltpu.SemaphoreType.DMA,
    ],
    compiler_params=pltpu.CompilerParams(
        use_tc_tiling_on_sc=True, needs_layout_passes=False),
)
def _fmt(xt_hbm, out_hbm, buf, sem):
    wid = lax.axis_index("s") * NC + lax.axis_index("c")
    c0 = wid * COLS
    pltpu.sync_copy(xt_hbm.at[:, pl.ds(c0, COLS)], buf)
    for f in range(FIELDS):
        pltpu.async_copy(
            buf.at[f], out_hbm.at[pl.ds(f * BATCH + c0, COLS)], sem)
    for f in range(FIELDS):
        pltpu.make_async_copy(
            buf.at[f], out_hbm.at[pl.ds(f * BATCH + c0, COLS)], sem).wait()


NBLK = INP // 128


@functools.partial(
    pl.kernel,
    out_type=jax.ShapeDtypeStruct((INP * EMBED_DIM,), jnp.float32),
    mesh=_mesh,
    scratch_types=[
        pltpu.VMEM((EMBED_DIM, 128), jnp.float32),
        pltpu.VMEM((EMBED_DIM, 128), jnp.float32),
        pltpu.VMEM((4096,), jnp.float32),
        pltpu.VMEM((4096,), jnp.float32),
        pltpu.VMEM((32, 16), jnp.int32),
        pltpu.VMEM((32, 16), jnp.int32),
        pltpu.SemaphoreType.DMA,
        pltpu.SemaphoreType.DMA,
        pltpu.SemaphoreType.DMA,
        pltpu.SemaphoreType.DMA,
    ],
    compiler_params=pltpu.CompilerParams(
        use_tc_tiling_on_sc=True, needs_layout_passes=False),
)
def _tpose(tt_hbm, tail_hbm, out_hbm, buf0, buf1, dbuf0, dbuf1, mtab, dtab,
           si0, si1, so0, so1):
    wid = lax.axis_index("s") * NC + lax.axis_index("c")
    iota = lax.iota(jnp.int32, 16)
    for k in range(32):
        m = (k + iota) & 31
        mtab[k, :] = m
        dtab[k, :] = iota * 32 + m
    lvecs = [l0 * 16 + iota for l0 in range(8)]
    bufs = (buf0, buf1)
    dbufs = (dbuf0, dbuf1)
    si = (si0, si1)
    so = (so0, so1)
    npw = NBLK // NW
    lo = wid * npw

    def in_copy(c, b):
        return [pltpu.make_async_copy(
            tt_hbm.at[pl.ds(e4 * 8, 8), pl.ds(c * 128, 128)],
            bufs[b].at[pl.ds(e4 * 8, 8)], si[b]) for e4 in range(4)]

    def out_copy(c, b):
        return pltpu.make_async_copy(
            dbufs[b], out_hbm.at[pl.ds(c * 4096, 4096)], so[b])

    def transpose_block(b, ncol16):
        def kbody(k, _):
            m = mtab[k, :]
            d = dtab[k, :]
            for l0 in range(ncol16):
                v = plsc.load_gather(bufs[b], [m, lvecs[l0]])
                plsc.store_scatter(dbufs[b], [d + l0 * 512], v)
            return 0
        lax.fori_loop(0, 32, kbody, 0)

    for cp in in_copy(lo, 0):
        cp.start()

    def body(j, _):
        for b in range(2):
            c = lo + 2 * j + b
            for cp in in_copy(c, b):
                cp.wait()

            @pl.when(2 * j + b + 1 < npw)
            def _():
                for cp in in_copy(c + 1, 1 - b):
                    cp.start()
            @pl.when(2 * j + b >= 2)
            def _():
                out_copy(c - 2, b).wait()
            transpose_block(b, 8)
            out_copy(c, b).start()
        return 0

    lax.fori_loop(0, npw // 2, body, 0)
    out_copy(lo + npw - 2, (npw - 2) % 2).wait()
    out_copy(lo + npw - 1, (npw - 1) % 2).wait()

    @pl.when(wid >= NW - 4)
    def _():
        c = NW * npw + (wid - (NW - 4))
        for cp in in_copy(c, 0):
            cp.start()
        for cp in in_copy(c, 0):
            cp.wait()
        transpose_block(0, 8)
        out_copy(c, 0).start()
        out_copy(c, 0).wait()

    @pl.when(wid == NW - 1)
    def _():
        pltpu.sync_copy(tail_hbm, dbuf0.at[pl.ds(0, 2048)])
        pltpu.sync_copy(dbuf0.at[pl.ds(0, 2048)],
                        out_hbm.at[pl.ds(NBLK * 4096, 2048)])


@functools.partial(
    pl.kernel,
    out_type=jax.ShapeDtypeStruct((B, EMBED_DIM), jnp.float32),
    mesh=_mesh,
    scratch_types=[
        pltpu.VMEM((2, CH), jnp.int32),
        pltpu.VMEM((2, CH, EMBED_DIM), jnp.float32),
        pltpu.SemaphoreType.DMA,
        pltpu.SemaphoreType.DMA,
        pltpu.SemaphoreType.DMA,
        pltpu.SemaphoreType.DMA,
        pltpu.SemaphoreType.DMA,
        pltpu.SemaphoreType.DMA,
    ],
    compiler_params=pltpu.CompilerParams(use_tc_tiling_on_sc=False),
)
def _embed(table_hbm, idx_hbm, out_hbm, idx_v, rows_v,
           si0, si1, sg0, sg1, ss0, ss1):
    wid = lax.axis_index("s") * NC + lax.axis_index("c")
    base = wid * BPW
    si = (si0, si1)
    sg = (sg0, sg1)
    ss = (ss0, ss1)

    def idx_copy(i):
        b = i % 2
        return pltpu.make_async_copy(
            idx_hbm.at[pl.ds(base + i * CH, CH)], idx_v.at[b], si[b])

    def gather_copy(i):
        b = i % 2
        return pltpu.make_async_copy(
            table_hbm.at[idx_v.at[b]], rows_v.at[b], sg[b])

    def store_copy(i):
        b = i % 2
        return pltpu.make_async_copy(
            rows_v.at[b], out_hbm.at[pl.ds(base + i * CH, CH)], ss[b])

    idx_copy(0).start()
    idx_copy(1).start()
    for i in range(NCH):
        idx_copy(i).wait()
        if i >= 2:
            store_copy(i - 2).wait()
        gather_copy(i).start()
        if i >= 1:
            gather_copy(i - 1).wait()
            store_copy(i - 1).start()
            if 2 <= i + 1 < NCH:
                idx_copy(i + 1).start()
    gather_copy(NCH - 1).wait()
    store_copy(NCH - 1).start()
    store_copy(NCH - 2).wait()
    store_copy(NCH - 1).wait()


def kernel(x, table):
    tail = table[INP - 64:].reshape(-1)
    tlin = _tpose(table.T, tail)
    tbl = tlin.reshape(INP, EMBED_DIM)
    idx = _fmt(x.T.astype(jnp.int32))
    out = _embed(tbl, idx)
    return out.reshape(FIELDS, BATCH, EMBED_DIM).transpose(1, 0, 2)
